# baseline (speedup 1.0000x reference)
import jax
import jax.numpy as jnp
from jax import lax
from jax.experimental import pallas as pl
from jax.experimental.pallas import tpu as pltpu
from jax.experimental.pallas import tpu_sc as plsc

_NC, _NS = 2, 16
_NW = _NC * _NS
_CHUNK = 32


def _sc_copy(pe_hbm, out_hbm, buf0, buf1, isem0, isem1, osem0, osem1):
    rows = out_hbm.shape[0]
    rows_per_w = rows // _NW
    nchunks = rows_per_w // _CHUNK
    wid = lax.axis_index("s") * _NC + lax.axis_index("c")
    base = wid * rows_per_w
    bufs = (buf0, buf1)
    isems = (isem0, isem1)
    osems = (osem0, osem1)
    ins, outs = [], []
    for i in range(nchunks):
        src = pe_hbm.at[pl.ds(base + i * _CHUNK, _CHUNK)]
        dst = out_hbm.at[pl.ds(base + i * _CHUNK, _CHUNK)]
        ins.append(pltpu.make_async_copy(src, bufs[i % 2], isems[i % 2]))
        outs.append(pltpu.make_async_copy(bufs[i % 2], dst, osems[i % 2]))
    ins[0].start()
    for i in range(nchunks):
        ins[i].wait()
        if i + 1 < nchunks:
            if i >= 1:
                outs[i - 1].wait()
            ins[i + 1].start()
        outs[i].start()
    outs[nchunks - 2].wait()
    outs[nchunks - 1].wait()


def kernel(x, pe):
    seq_len = x.shape[1]
    d = pe.shape[1]
    run = pl.kernel(
        _sc_copy,
        out_type=jax.ShapeDtypeStruct((seq_len, d), pe.dtype),
        mesh=plsc.VectorSubcoreMesh(
            core_axis_name="c", subcore_axis_name="s",
            num_cores=_NC, num_subcores=_NS,
        ),
        scratch_types=[
            pltpu.VMEM((_CHUNK, d), pe.dtype),
            pltpu.VMEM((_CHUNK, d), pe.dtype),
            pltpu.SemaphoreType.DMA,
            pltpu.SemaphoreType.DMA,
            pltpu.SemaphoreType.DMA,
            pltpu.SemaphoreType.DMA,
        ],
    )
    return run(pe)[None]


# speedup vs baseline: 2.9946x; 2.9946x over previous
import jax
import jax.numpy as jnp
from jax.experimental import pallas as pl
from jax.experimental.pallas import tpu as pltpu

_NCHUNK = 4


def _dma_copy(pe_any, out_any, bufs, isems, osems):
    rows = out_any.shape[0]
    chunk = rows // _NCHUNK
    ins, outs = [], []
    for i in range(_NCHUNK):
        ins.append(pltpu.make_async_copy(
            pe_any.at[pl.ds(i * chunk, chunk)], bufs.at[i], isems.at[i]))
        outs.append(pltpu.make_async_copy(
            bufs.at[i], out_any.at[pl.ds(i * chunk, chunk)], osems.at[i]))
    for c in ins:
        c.start()
    for i in range(_NCHUNK):
        ins[i].wait()
        outs[i].start()
    for c in outs:
        c.wait()


def kernel(x, pe):
    seq_len = x.shape[1]
    d = pe.shape[1]
    out = pl.pallas_call(
        _dma_copy,
        in_specs=[pl.BlockSpec(memory_space=pl.ANY)],
        out_specs=pl.BlockSpec(memory_space=pl.ANY),
        out_shape=jax.ShapeDtypeStruct((seq_len, d), pe.dtype),
        scratch_shapes=[
            pltpu.VMEM((_NCHUNK, seq_len // _NCHUNK, d), pe.dtype),
            pltpu.SemaphoreType.DMA((_NCHUNK,)),
            pltpu.SemaphoreType.DMA((_NCHUNK,)),
        ],
    )(pe)
    return out[None]
